# baseline (speedup 1.0000x reference)
import jax
import jax.numpy as jnp
from jax.experimental import pallas as pl
from jax.experimental.pallas import tpu as pltpu

B = 1024
L = 256
NP = 10
D = 200
NT = 4

IB = 256
DB = 40


def _body(tid_ref, patch_ref, prompt_ref, out_ref):
    x = patch_ref[...]
    for d in range(DB):
        out_ref[NP:, d, :] = x[:, d, :].T
    tid = tid_ref[0, 0, :]
    tab = prompt_ref[...]
    acc = jnp.zeros((NP, DB, IB), jnp.float32)
    for t in range(NT):
        cand = jnp.transpose(tab[t], (1, 0))[:, :, None]
        mask = (tid == t)[None, None, :]
        acc = jnp.where(mask, cand, acc)
    out_ref[:NP, :, :] = acc


@jax.jit
def _concat(task_id, patch_embeddings, prompt_tokens):
    patch_t = jnp.transpose(patch_embeddings, (0, 2, 1))
    prompt_t = jnp.transpose(prompt_tokens, (0, 2, 1))
    tid2 = task_id.reshape(B // IB, 1, IB)
    fn = pl.pallas_call(
        _body,
        grid=(B // IB, D // DB),
        in_specs=[
            pl.BlockSpec((1, 1, IB), lambda ib, db: (ib, 0, 0)),
            pl.BlockSpec((IB, DB, L), lambda ib, db: (ib, db, 0)),
            pl.BlockSpec((NT, DB, NP), lambda ib, db: (0, db, 0)),
        ],
        out_specs=pl.BlockSpec((NP + L, DB, IB), lambda ib, db: (0, db, ib)),
        out_shape=jax.ShapeDtypeStruct((NP + L, D, B), jnp.float32),
        compiler_params=pltpu.CompilerParams(
            dimension_semantics=("parallel", "parallel")),
    )
    out_t = fn(tid2, patch_t, prompt_t)
    return jnp.transpose(out_t, (2, 0, 1))


def kernel(task_id, patch_embeddings, prompt_tokens):
    return _concat(task_id.astype(jnp.int32), patch_embeddings,
                   prompt_tokens)
